# Initial kernel scaffold; baseline (speedup 1.0000x reference)
import jax
import jax.numpy as jnp
from jax.experimental import pallas as pl


def kernel(x, table):
    raise NotImplementedError("write your pallas kernel here")


# speedup vs baseline: 4.6150x; 4.6150x over previous
import jax
import jax.numpy as jnp
from jax import lax
from jax.experimental import pallas as pl
from jax.experimental.pallas import tpu as pltpu
from jax.experimental.pallas import tpu_sc as plsc

VOCAB = 100000
EMBED_DIM = 64
BATCH = 4096
HIST_LEN = 50
NC, NS = 2, 16
NW = NC * NS
BPW = BATCH // NW
G = 16
NG = BPW // G


def _gather_body(x_hbm, table_hbm, out_hbm, idx_v, stage, gsem, osem):
    wid = lax.axis_index("s") * NC + lax.axis_index("c")
    b0 = wid * BPW
    pltpu.sync_copy(x_hbm.at[pl.ds(b0, BPW)], idx_v)

    def fire_gathers(g, h):
        for i in range(G):
            pltpu.async_copy(table_hbm.at[idx_v.at[g * G + i]],
                             stage.at[h, i], gsem)

    def drain_gathers(g, h):
        for i in range(G):
            pltpu.make_async_copy(table_hbm.at[idx_v.at[g * G + i]],
                                  stage.at[h, i], gsem).wait()

    def fire_ocopy(g, h):
        pltpu.async_copy(stage.at[h], out_hbm.at[pl.ds(b0 + g * G, G)], osem)

    def drain_ocopy(g, h):
        pltpu.make_async_copy(stage.at[h], out_hbm.at[pl.ds(b0 + g * G, G)],
                              osem).wait()

    fire_gathers(0, 0)

    def body(g, carry):
        h = lax.rem(g, 2)

        drain_gathers(g, h)

        @pl.when(g >= 1)
        def _():
            drain_ocopy(g - 1, 1 - h)

        @pl.when(g < NG - 1)
        def _():
            fire_gathers(g + 1, 1 - h)

        fire_ocopy(g, h)
        return carry

    lax.fori_loop(0, NG, body, 0)
    drain_ocopy(NG - 1, (NG - 1) % 2)


def kernel(x, table):
    mesh = plsc.VectorSubcoreMesh(core_axis_name="c", subcore_axis_name="s")
    k = pl.kernel(
        _gather_body,
        mesh=mesh,
        out_type=jax.ShapeDtypeStruct((BATCH, HIST_LEN, EMBED_DIM),
                                      jnp.float32),
        scratch_types=[
            pltpu.VMEM((BPW, HIST_LEN), jnp.int32),
            pltpu.VMEM((2, G, HIST_LEN, EMBED_DIM), jnp.float32),
            pltpu.SemaphoreType.DMA,
            pltpu.SemaphoreType.DMA,
        ],
        compiler_params=pltpu.CompilerParams(use_tc_tiling_on_sc=False),
    )
    return k(x, table)
